# baseline (speedup 1.0000x reference)
import jax
import jax.numpy as jnp
from jax.experimental import pallas as pl

IMAGES = 4
P = 2000
G = 100
MH = 56
MW = 56
TRAIN_ROIS = 200
POS_COUNT = int(TRAIN_ROIS * 0.33)
NEG_COUNT = TRAIN_ROIS - POS_COUNT
MS = 28
EPS = 1e-6


def _first_max_index(score, iota_p, sentinel):
    maxv = jnp.max(score)
    return jnp.min(jnp.where(score == maxv, iota_p, sentinel))


def _dtl_kernel(prop_ref, cid_ref, gtb_ref, gtm_ref, rnd_ref,
                rois_ref, cls_ref, dlt_ref, msk_ref):
    prop = prop_ref[0]
    gtb = gtb_ref[0]
    cid = cid_ref[0]
    rnd = rnd_ref[0]

    msk_ref[...] = jnp.zeros((1, TRAIN_ROIS, MS, MS), jnp.float32)

    py1 = prop[:, 0:1]
    px1 = prop[:, 1:2]
    py2 = prop[:, 2:3]
    px2 = prop[:, 3:4]
    gy1 = gtb[:, 0].reshape(1, G)
    gx1 = gtb[:, 1].reshape(1, G)
    gy2 = gtb[:, 2].reshape(1, G)
    gx2 = gtb[:, 3].reshape(1, G)

    iy1 = jnp.maximum(py1, gy1)
    ix1 = jnp.maximum(px1, gx1)
    iy2 = jnp.minimum(py2, gy2)
    ix2 = jnp.minimum(px2, gx2)
    inter = jnp.maximum(iy2 - iy1, 0.0) * jnp.maximum(ix2 - ix1, 0.0)
    a_p = (py2 - py1) * (px2 - px1)
    a_g = (gy2 - gy1) * (gx2 - gx1)
    union = a_p + a_g - inter
    iou = inter / jnp.maximum(union, EPS)

    gt_valid = (cid > 0) & (a_g > 0.0)
    ovl = jnp.where(gt_valid, iou, -1.0)

    roi_max = jnp.max(ovl, axis=1, keepdims=True)
    iota_pg = jax.lax.broadcasted_iota(jnp.int32, (P, G), 1)
    assign_col = jnp.min(jnp.where(ovl == roi_max, iota_pg, G), axis=1,
                         keepdims=True)
    assign_all = jnp.transpose(assign_col)
    pos = jnp.transpose(roi_max >= 0.5)

    pos_score = jnp.where(pos, rnd, -1.0)
    neg_score = jnp.where(pos, -1.0, rnd)

    iota_p = jax.lax.broadcasted_iota(jnp.int32, (1, P), 1)
    iota_g = jax.lax.broadcasted_iota(jnp.int32, (1, G), 1)
    iota_t = jax.lax.broadcasted_iota(jnp.int32, (1, TRAIN_ROIS), 1)
    rows_t = jax.lax.broadcasted_iota(jnp.int32, (TRAIN_ROIS, 1), 0)
    iota_4 = jax.lax.broadcasted_iota(jnp.int32, (1, 4), 1)
    col_h = jax.lax.broadcasted_iota(jnp.int32, (MS, MH), 1)
    col_w = jax.lax.broadcasted_iota(jnp.int32, (MS, MW), 1)
    tcol = jax.lax.broadcasted_iota(jnp.int32, (MS, 1), 0).astype(
        jnp.float32) / (MS - 1)

    def pos_body(i, carry):
        score, rois, cls, dlt = carry
        idx = _first_max_index(score, iota_p, P)
        m = iota_p == idx
        mp = jnp.transpose(m)
        row = jnp.sum(jnp.where(mp, prop, 0.0), axis=0, keepdims=True)
        pvi = jnp.sum(jnp.where(m, pos.astype(jnp.int32), 0))
        pvf = pvi.astype(jnp.float32)
        a = jnp.sum(jnp.where(m, assign_all, 0))

        mg = iota_g == a
        mgt = jnp.transpose(mg)
        grow = jnp.sum(jnp.where(mgt, gtb, 0.0), axis=0, keepdims=True)
        clsval = jnp.sum(jnp.where(mg, cid, 0))

        y1 = row[0, 0]
        x1 = row[0, 1]
        y2 = row[0, 2]
        x2 = row[0, 3]
        by1 = grow[0, 0]
        bx1 = grow[0, 1]
        by2 = grow[0, 2]
        bx2 = grow[0, 3]

        h = jnp.maximum(y2 - y1, EPS)
        w = jnp.maximum(x2 - x1, EPS)
        cy = y1 + 0.5 * h
        cx = x1 + 0.5 * w
        gh = jnp.maximum(by2 - by1, EPS)
        gw = jnp.maximum(bx2 - bx1, EPS)
        gcy = by1 + 0.5 * gh
        gcx = bx1 + 0.5 * gw
        d0 = ((gcy - cy) / h) / 0.1 * pvf
        d1 = ((gcx - cx) / w) / 0.1 * pvf
        d2 = jnp.log(gh / h) / 0.2 * pvf
        d3 = jnp.log(gw / w) / 0.2 * pvf
        drow = (jnp.where(iota_4 == 0, d0, 0.0) + jnp.where(iota_4 == 1, d1, 0.0)
                + jnp.where(iota_4 == 2, d2, 0.0) + jnp.where(iota_4 == 3, d3, 0.0))

        ry1 = (y1 - by1) / gh
        rx1 = (x1 - bx1) / gw
        ry2 = (y2 - by1) / gh
        rx2 = (x2 - bx1) / gw
        ys = ry1 * (MH - 1) + tcol * (ry2 - ry1) * (MH - 1)
        xs = rx1 * (MW - 1) + tcol * (rx2 - rx1) * (MW - 1)
        y0f = jnp.floor(ys)
        x0f = jnp.floor(xs)
        wy = ys - y0f
        wx = xs - x0f
        y0 = jnp.clip(y0f.astype(jnp.int32), 0, MH - 1)
        y1i = jnp.clip(y0 + 1, 0, MH - 1)
        x0 = jnp.clip(x0f.astype(jnp.int32), 0, MW - 1)
        x1i = jnp.clip(x0 + 1, 0, MW - 1)
        wmy = (jnp.where(col_h == y0, 1.0 - wy, 0.0)
               + jnp.where(col_h == y1i, wy, 0.0))
        wmx = (jnp.where(col_w == x0, 1.0 - wx, 0.0)
               + jnp.where(col_w == x1i, wx, 0.0))
        img = gtm_ref[0, pl.ds(a, 1), :, :][0]
        tmp = jax.lax.dot_general(img, wmx, (((1,), (1,)), ((), ())),
                                  preferred_element_type=jnp.float32,
                                  precision=jax.lax.Precision.HIGHEST)
        crop = jnp.dot(wmy, tmp, preferred_element_type=jnp.float32,
                       precision=jax.lax.Precision.HIGHEST)
        mrow = jnp.round(crop) * pvf
        msk_ref[0, pl.ds(i, 1), :, :] = mrow.reshape(1, MS, MS)

        rois = jnp.where(rows_t == i, row, rois)
        cls = jnp.where(iota_t == i, clsval * pvi, cls)
        dlt = jnp.where(rows_t == i, drow, dlt)
        score = jnp.where(m, -3.0, score)
        return score, rois, cls, dlt

    rois0 = jnp.zeros((TRAIN_ROIS, 4), jnp.float32)
    cls0 = jnp.zeros((1, TRAIN_ROIS), jnp.int32)
    dlt0 = jnp.zeros((TRAIN_ROIS, 4), jnp.float32)
    pos_score, rois, cls, dlt = jax.lax.fori_loop(
        0, POS_COUNT, pos_body, (pos_score, rois0, cls0, dlt0))

    def neg_body(j, carry):
        score, rois = carry
        idx = _first_max_index(score, iota_p, P)
        m = iota_p == idx
        mp = jnp.transpose(m)
        row = jnp.sum(jnp.where(mp, prop, 0.0), axis=0, keepdims=True)
        rois = jnp.where(rows_t == (POS_COUNT + j), row, rois)
        score = jnp.where(m, -3.0, score)
        return score, rois

    _, rois = jax.lax.fori_loop(0, NEG_COUNT, neg_body, (neg_score, rois))

    rois_ref[0] = rois
    cls_ref[0] = cls
    dlt_ref[0] = dlt


@jax.jit
def kernel(proposals, prior_class_ids, prior_boxes, prior_masks):
    keys = jax.random.split(jax.random.key(42), IMAGES)
    rnd = jax.vmap(lambda k: jax.random.uniform(k, (P,), dtype=jnp.float32))(keys)
    masks_f = jnp.transpose(prior_masks, (0, 3, 1, 2)).astype(jnp.float32)
    cids = prior_class_ids.astype(jnp.int32)
    rois, cls, dlt, msk = pl.pallas_call(
        _dtl_kernel,
        grid=(IMAGES,),
        in_specs=[
            pl.BlockSpec((1, P, 4), lambda i: (i, 0, 0)),
            pl.BlockSpec((1, 1, G), lambda i: (i, 0, 0)),
            pl.BlockSpec((1, G, 4), lambda i: (i, 0, 0)),
            pl.BlockSpec((1, G, MH, MW), lambda i: (i, 0, 0, 0)),
            pl.BlockSpec((1, 1, P), lambda i: (i, 0, 0)),
        ],
        out_specs=[
            pl.BlockSpec((1, TRAIN_ROIS, 4), lambda i: (i, 0, 0)),
            pl.BlockSpec((1, 1, TRAIN_ROIS), lambda i: (i, 0, 0)),
            pl.BlockSpec((1, TRAIN_ROIS, 4), lambda i: (i, 0, 0)),
            pl.BlockSpec((1, TRAIN_ROIS, MS, MS), lambda i: (i, 0, 0, 0)),
        ],
        out_shape=[
            jax.ShapeDtypeStruct((IMAGES, TRAIN_ROIS, 4), jnp.float32),
            jax.ShapeDtypeStruct((IMAGES, 1, TRAIN_ROIS), jnp.int32),
            jax.ShapeDtypeStruct((IMAGES, TRAIN_ROIS, 4), jnp.float32),
            jax.ShapeDtypeStruct((IMAGES, TRAIN_ROIS, MS, MS), jnp.float32),
        ],
    )(proposals, cids.reshape(IMAGES, 1, G), prior_boxes, masks_f,
      rnd.reshape(IMAGES, 1, P))
    return (rois, cls.reshape(IMAGES, TRAIN_ROIS).astype(prior_class_ids.dtype),
            dlt, msk)
